# baseline (speedup 1.0000x reference)
import functools

import jax
import jax.numpy as jnp
from jax import lax
from jax.experimental import pallas as pl
from jax.experimental.pallas import tpu as pltpu
from jax.experimental.pallas import tpu_sc as plsc

N = 16384
H = 64
HH = 32
NNZ = 2684354
NS = 16
EW = 128
CH = 4
RPT = 1312
NSTEP = RPT // CH
NNZ_PAD = NS * RPT * EW
RT_OUT = N // NS
PG = 8
BLK = 1024


def _theta_diff_body(p_ref, lp_ref, w2t_ref, w3t_ref, bd_ref, out_ref):
    y = (
        jnp.dot(p_ref[...], w2t_ref[...], preferred_element_type=jnp.float32)
        - jnp.dot(lp_ref[...], w3t_ref[...], preferred_element_type=jnp.float32)
        + bd_ref[...]
    )
    out_ref[0] = y[:, :HH]
    out_ref[1] = y[:, HH:]


def _theta_diff(primal, last_primal, w2t, w3t, bd):
    return pl.pallas_call(
        _theta_diff_body,
        grid=(N // BLK,),
        in_specs=[
            pl.BlockSpec((BLK, H), lambda i: (i, 0)),
            pl.BlockSpec((BLK, H), lambda i: (i, 0)),
            pl.BlockSpec((H, H), lambda i: (0, 0)),
            pl.BlockSpec((H, H), lambda i: (0, 0)),
            pl.BlockSpec((1, H), lambda i: (0, 0)),
        ],
        out_specs=pl.BlockSpec((2, BLK, HH), lambda i: (0, i, 0)),
        out_shape=jax.ShapeDtypeStruct((2, N, HH), jnp.float32),
    )(primal, last_primal, w2t, w3t, bd)


def _final_body(d_ref, rhs_ref, sp_ref, w1t_ref, b1_ref, sig_ref, out_ref):
    y = (
        jnp.dot(d_ref[...], w1t_ref[...], preferred_element_type=jnp.float32)
        + b1_ref[...]
    )
    s = jnp.concatenate([sp_ref[0], sp_ref[1]], axis=1)
    y = y + sig_ref[0] * (s - rhs_ref[...])
    out_ref[...] = jnp.where(y >= 0, y, 0.01 * y)


def _final(dual, rhs, spmm2, w1t, b1, sig):
    return pl.pallas_call(
        _final_body,
        grid=(N // BLK,),
        in_specs=[
            pl.BlockSpec((BLK, H), lambda i: (i, 0)),
            pl.BlockSpec((BLK, H), lambda i: (i, 0)),
            pl.BlockSpec((2, BLK, HH), lambda i: (0, i, 0)),
            pl.BlockSpec((H, H), lambda i: (0, 0)),
            pl.BlockSpec((1, H), lambda i: (0, 0)),
            pl.BlockSpec(memory_space=pltpu.SMEM),
        ],
        out_specs=pl.BlockSpec((BLK, H), lambda i: (i, 0)),
        out_shape=jax.ShapeDtypeStruct((N, H), jnp.float32),
    )(dual, rhs, spmm2, w1t, b1, sig)


def _sc_spmm_body(
    xs_hbm, cols_hbm, rows_hbm, vals_hbm, out_hbm,
    xs_sh, acc_sh, gb0, gb1, t_v, cols_v, rows_v, vals_v, gsem, ssem,
):
    c = lax.axis_index("c")
    s = lax.axis_index("s")
    row0 = s * RT_OUT

    pltpu.sync_copy(
        xs_hbm.at[c, pl.ds(row0, RT_OUT)], xs_sh.at[pl.ds(row0, RT_OUT)]
    )

    zeros16 = jnp.zeros((16,), jnp.float32)

    def _zero_row(r, carry):
        for q in range(CH):
            t_v[q, r, pl.ds(0, 16)] = zeros16
            t_v[q, r, pl.ds(16, 16)] = zeros16
        return carry

    lax.fori_loop(0, EW, _zero_row, 0)
    for k in range(RT_OUT // EW):
        pltpu.sync_copy(t_v.at[k % CH], acc_sh.at[pl.ds(row0 + k * EW, EW)])
    plsc.subcore_barrier()

    iota16 = lax.iota(jnp.int32, 16)
    p_vecs = [jnp.full((16,), p, jnp.int32) for p in range(HH)]

    def _stage_idx(m):
        pltpu.sync_copy(cols_hbm.at[s, pl.ds(m * CH, CH)], cols_v)
        pltpu.sync_copy(rows_hbm.at[s, pl.ds(m * CH, CH)], rows_v)
        pltpu.sync_copy(vals_hbm.at[s, pl.ds(m * CH, CH)], vals_v)

    def _step(src_gb, dst_gb, m, fire_pred):
        for r in range(CH):
            pltpu.make_async_copy(
                xs_sh.at[cols_v.at[r]], src_gb.at[r], gsem
            ).wait()

        @pl.when(fire_pred)
        def _fire_gathers():
            pltpu.sync_copy(cols_hbm.at[s, pl.ds((m + 1) * CH, CH)], cols_v)
            for r in range(CH):
                pltpu.async_copy(
                    xs_sh.at[cols_v.at[r]], dst_gb.at[r], gsem
                )
        for r in range(CH):
            gsrc = src_gb.at[r]
            tdst = t_v.at[r]

            @plsc.parallel_loop(0, EW // 16, 1, unroll=2)
            def _mul16(k, gsrc=gsrc, tdst=tdst, r=r):
                e_vec = iota16 + k * 16
                v16 = vals_v[r, pl.ds(k * 16, 16)]
                for p0 in range(0, HH, PG):
                    cols = [
                        plsc.load_gather(gsrc, [e_vec, p_vecs[p0 + q]])
                        for q in range(PG)
                    ]
                    prods = [col * v16 for col in cols]
                    for q in range(PG):
                        plsc.store_scatter(
                            tdst, [e_vec, p_vecs[p0 + q]], prods[q]
                        )

            pltpu.async_copy(t_v.at[r], acc_sh.at[rows_v.at[r]], ssem, add=True)
        for r in range(CH):
            pltpu.make_async_copy(
                t_v.at[r], acc_sh.at[rows_v.at[r]], ssem
            ).wait()
        @pl.when(fire_pred)
        def _stage_next():
            pltpu.sync_copy(rows_hbm.at[s, pl.ds((m + 1) * CH, CH)], rows_v)
            pltpu.sync_copy(vals_hbm.at[s, pl.ds((m + 1) * CH, CH)], vals_v)

    _stage_idx(0)
    for r in range(CH):
        pltpu.async_copy(xs_sh.at[cols_v.at[r]], gb0.at[r], gsem)

    def _two_steps(m2, carry):
        m = m2 * 2
        _step(gb0, gb1, m, jnp.bool_(True))
        _step(gb1, gb0, m + 1, m + 2 < NSTEP)
        return carry

    lax.fori_loop(0, NSTEP // 2, _two_steps, 0)

    plsc.subcore_barrier()
    pltpu.sync_copy(
        acc_sh.at[pl.ds(row0, RT_OUT)], out_hbm.at[c, pl.ds(row0, RT_OUT)]
    )


@functools.cache
def _sc_spmm():
    return pl.kernel(
        _sc_spmm_body,
        out_type=jax.ShapeDtypeStruct((2, N, HH), jnp.float32),
        mesh=plsc.VectorSubcoreMesh(core_axis_name="c", subcore_axis_name="s"),
        compiler_params=pltpu.CompilerParams(
            needs_layout_passes=False, use_tc_tiling_on_sc=False
        ),
        scratch_types=[
            pltpu.VMEM_SHARED((N, HH), jnp.float32),
            pltpu.VMEM_SHARED((N, HH), jnp.float32),
            pltpu.VMEM((CH, EW, HH), jnp.float32),
            pltpu.VMEM((CH, EW, HH), jnp.float32),
            pltpu.VMEM((CH, EW, HH), jnp.float32),
            pltpu.VMEM((CH, EW), jnp.int32),
            pltpu.VMEM((CH, EW), jnp.int32),
            pltpu.VMEM((CH, EW), jnp.float32),
            pltpu.SemaphoreType.DMA,
            pltpu.SemaphoreType.DMA,
        ],
    )


def kernel(primal, last_primal, dual, cons_indices, cons_values,
           right_hand_side, W1, b1, W2, b2, W3, b3, sigma):
    rows = cons_indices[0]
    cols = cons_indices[1]
    pad = NNZ_PAD - NNZ
    cols3 = jnp.pad(cols, (0, pad)).reshape(NS, RPT, EW)
    rows3 = jnp.pad(rows, (0, pad)).reshape(NS, RPT, EW)
    vals3 = jnp.pad(cons_values, (0, pad)).reshape(NS, RPT, EW)

    xs = _theta_diff(primal, last_primal, W2.T, W3.T, (b2 - b3).reshape(1, H))
    spmm2 = _sc_spmm()(xs, cols3, rows3, vals3)
    return _final(
        dual, right_hand_side, spmm2, W1.T, b1.reshape(1, H), sigma.reshape(1)
    )


# speedup vs baseline: 1.7907x; 1.7907x over previous
import functools

import jax
import jax.numpy as jnp
from jax import lax
from jax.experimental import pallas as pl
from jax.experimental.pallas import tpu as pltpu
from jax.experimental.pallas import tpu_sc as plsc

N = 16384
H = 64
HH = 32
NNZ = 2684354
NS = 16
EW = 128
CH = 4
RPT = 1312
NSTEP = RPT // CH
NNZ_PAD = NS * RPT * EW
RT_OUT = N // NS
PG = 8
BLK = 1024


def _theta_diff_body(p_ref, lp_ref, w2t_ref, w3t_ref, bd_ref, out_ref):
    y = (
        jnp.dot(p_ref[...], w2t_ref[...], preferred_element_type=jnp.float32)
        - jnp.dot(lp_ref[...], w3t_ref[...], preferred_element_type=jnp.float32)
        + bd_ref[...]
    )
    out_ref[0] = y[:, :HH]
    out_ref[1] = y[:, HH:]


def _theta_diff(primal, last_primal, w2t, w3t, bd):
    return pl.pallas_call(
        _theta_diff_body,
        grid=(N // BLK,),
        in_specs=[
            pl.BlockSpec((BLK, H), lambda i: (i, 0)),
            pl.BlockSpec((BLK, H), lambda i: (i, 0)),
            pl.BlockSpec((H, H), lambda i: (0, 0)),
            pl.BlockSpec((H, H), lambda i: (0, 0)),
            pl.BlockSpec((1, H), lambda i: (0, 0)),
        ],
        out_specs=pl.BlockSpec((2, BLK, HH), lambda i: (0, i, 0)),
        out_shape=jax.ShapeDtypeStruct((2, N, HH), jnp.float32),
    )(primal, last_primal, w2t, w3t, bd)


def _final_body(d_ref, rhs_ref, sp_ref, w1t_ref, b1_ref, sig_ref, out_ref):
    y = (
        jnp.dot(d_ref[...], w1t_ref[...], preferred_element_type=jnp.float32)
        + b1_ref[...]
    )
    s = jnp.concatenate([sp_ref[0], sp_ref[1]], axis=1)
    y = y + sig_ref[0] * (s - rhs_ref[...])
    out_ref[...] = jnp.where(y >= 0, y, 0.01 * y)


def _final(dual, rhs, spmm2, w1t, b1, sig):
    return pl.pallas_call(
        _final_body,
        grid=(N // BLK,),
        in_specs=[
            pl.BlockSpec((BLK, H), lambda i: (i, 0)),
            pl.BlockSpec((BLK, H), lambda i: (i, 0)),
            pl.BlockSpec((2, BLK, HH), lambda i: (0, i, 0)),
            pl.BlockSpec((H, H), lambda i: (0, 0)),
            pl.BlockSpec((1, H), lambda i: (0, 0)),
            pl.BlockSpec(memory_space=pltpu.SMEM),
        ],
        out_specs=pl.BlockSpec((BLK, H), lambda i: (i, 0)),
        out_shape=jax.ShapeDtypeStruct((N, H), jnp.float32),
    )(dual, rhs, spmm2, w1t, b1, sig)


def _sc_spmm_body(
    xs_hbm, cols_hbm, rows_hbm, vals_hbm, out_hbm,
    xs_sh, acc_sh, gb0, gb1, t_v, cols_v, rows_v, vals_v, gsem, ssem,
):
    c = lax.axis_index("c")
    s = lax.axis_index("s")
    row0 = s * RT_OUT

    pltpu.sync_copy(
        xs_hbm.at[c, pl.ds(row0, RT_OUT)], xs_sh.at[pl.ds(row0, RT_OUT)]
    )

    zeros16 = jnp.zeros((16,), jnp.float32)

    def _zero_row(r, carry):
        for q in range(CH):
            t_v[q, r, pl.ds(0, 16)] = zeros16
            t_v[q, r, pl.ds(16, 16)] = zeros16
        return carry

    lax.fori_loop(0, EW, _zero_row, 0)
    for k in range(RT_OUT // EW):
        pltpu.sync_copy(t_v.at[k % CH], acc_sh.at[pl.ds(row0 + k * EW, EW)])
    plsc.subcore_barrier()


    i_consts = [jnp.full((16,), i, jnp.int32) for i in range(16)]

    def _stage_idx(m):
        pltpu.sync_copy(cols_hbm.at[s, pl.ds(m * CH, CH)], cols_v)
        pltpu.sync_copy(rows_hbm.at[s, pl.ds(m * CH, CH)], rows_v)
        pltpu.sync_copy(vals_hbm.at[s, pl.ds(m * CH, CH)], vals_v)

    def _step(src_gb, dst_gb, m, fire_pred):
        for r in range(CH):
            pltpu.make_async_copy(
                xs_sh.at[cols_v.at[r]], src_gb.at[r], gsem
            ).wait()

        @pl.when(fire_pred)
        def _fire_gathers():
            pltpu.sync_copy(cols_hbm.at[s, pl.ds((m + 1) * CH, CH)], cols_v)
            for r in range(CH):
                pltpu.async_copy(
                    xs_sh.at[cols_v.at[r]], dst_gb.at[r], gsem
                )
        for r in range(CH):
            vrow = vals_v.at[r]

            @plsc.parallel_loop(0, EW // 16, 1, unroll=2)
            def _mul16(k, vrow=vrow, r=r):
                ev = jnp.full((16,), k * 16, jnp.int32)
                for i in range(16):
                    e = k * 16 + i
                    vb = plsc.load_gather(vrow, [ev + i_consts[i]])
                    for h in range(HH // 16):
                        t_v[r, e, pl.ds(h * 16, 16)] = (
                            src_gb[r, e, pl.ds(h * 16, 16)] * vb
                        )

            pltpu.async_copy(t_v.at[r], acc_sh.at[rows_v.at[r]], ssem, add=True)
        for r in range(CH):
            pltpu.make_async_copy(
                t_v.at[r], acc_sh.at[rows_v.at[r]], ssem
            ).wait()
        @pl.when(fire_pred)
        def _stage_next():
            pltpu.sync_copy(rows_hbm.at[s, pl.ds((m + 1) * CH, CH)], rows_v)
            pltpu.sync_copy(vals_hbm.at[s, pl.ds((m + 1) * CH, CH)], vals_v)

    _stage_idx(0)
    for r in range(CH):
        pltpu.async_copy(xs_sh.at[cols_v.at[r]], gb0.at[r], gsem)

    def _two_steps(m2, carry):
        m = m2 * 2
        _step(gb0, gb1, m, jnp.bool_(True))
        _step(gb1, gb0, m + 1, m + 2 < NSTEP)
        return carry

    lax.fori_loop(0, NSTEP // 2, _two_steps, 0)

    plsc.subcore_barrier()
    pltpu.sync_copy(
        acc_sh.at[pl.ds(row0, RT_OUT)], out_hbm.at[c, pl.ds(row0, RT_OUT)]
    )


@functools.cache
def _sc_spmm():
    return pl.kernel(
        _sc_spmm_body,
        out_type=jax.ShapeDtypeStruct((2, N, HH), jnp.float32),
        mesh=plsc.VectorSubcoreMesh(core_axis_name="c", subcore_axis_name="s"),
        compiler_params=pltpu.CompilerParams(
            needs_layout_passes=False, use_tc_tiling_on_sc=False
        ),
        scratch_types=[
            pltpu.VMEM_SHARED((N, HH), jnp.float32),
            pltpu.VMEM_SHARED((N, HH), jnp.float32),
            pltpu.VMEM((CH, EW, HH), jnp.float32),
            pltpu.VMEM((CH, EW, HH), jnp.float32),
            pltpu.VMEM((CH, EW, HH), jnp.float32),
            pltpu.VMEM((CH, EW), jnp.int32),
            pltpu.VMEM((CH, EW), jnp.int32),
            pltpu.VMEM((CH, EW), jnp.float32),
            pltpu.SemaphoreType.DMA,
            pltpu.SemaphoreType.DMA,
        ],
    )


def kernel(primal, last_primal, dual, cons_indices, cons_values,
           right_hand_side, W1, b1, W2, b2, W3, b3, sigma):
    rows = cons_indices[0]
    cols = cons_indices[1]
    pad = NNZ_PAD - NNZ
    cols3 = jnp.pad(cols, (0, pad)).reshape(NS, RPT, EW)
    rows3 = jnp.pad(rows, (0, pad)).reshape(NS, RPT, EW)
    vals3 = jnp.pad(cons_values, (0, pad)).reshape(NS, RPT, EW)

    xs = _theta_diff(primal, last_primal, W2.T, W3.T, (b2 - b3).reshape(1, H))
    spmm2 = _sc_spmm()(xs, cols3, rows3, vals3)
    return _final(
        dual, right_hand_side, spmm2, W1.T, b1.reshape(1, H), sigma.reshape(1)
    )


# speedup vs baseline: 3.8235x; 2.1353x over previous
import functools

import jax
import jax.numpy as jnp
from jax import lax
from jax.experimental import pallas as pl
from jax.experimental.pallas import tpu as pltpu
from jax.experimental.pallas import tpu_sc as plsc

N = 16384
H = 64
HH = 32
NNZ = 2684354
NS = 16
SE = 512
RPT = 1312
NSTEP = RPT * 128 // SE
NNZ_PAD = NS * RPT * 128
RT_OUT = N // NS
BLK = 1024


def _theta_diff_body(p_ref, lp_ref, w2t_ref, w3t_ref, bd_ref, out_ref):
    y = (
        jnp.dot(p_ref[...], w2t_ref[...], preferred_element_type=jnp.float32)
        - jnp.dot(lp_ref[...], w3t_ref[...], preferred_element_type=jnp.float32)
        + bd_ref[...]
    )
    out_ref[0] = y[:, :HH]
    out_ref[1] = y[:, HH:]


def _theta_diff(primal, last_primal, w2t, w3t, bd):
    return pl.pallas_call(
        _theta_diff_body,
        grid=(N // BLK,),
        in_specs=[
            pl.BlockSpec((BLK, H), lambda i: (i, 0)),
            pl.BlockSpec((BLK, H), lambda i: (i, 0)),
            pl.BlockSpec((H, H), lambda i: (0, 0)),
            pl.BlockSpec((H, H), lambda i: (0, 0)),
            pl.BlockSpec((1, H), lambda i: (0, 0)),
        ],
        out_specs=pl.BlockSpec((2, BLK, HH), lambda i: (0, i, 0)),
        out_shape=jax.ShapeDtypeStruct((2, N, HH), jnp.float32),
    )(primal, last_primal, w2t, w3t, bd)


def _final_body(d_ref, rhs_ref, sp_ref, w1t_ref, b1_ref, sig_ref, out_ref):
    y = (
        jnp.dot(d_ref[...], w1t_ref[...], preferred_element_type=jnp.float32)
        + b1_ref[...]
    )
    s = jnp.concatenate([sp_ref[0], sp_ref[1]], axis=1)
    y = y + sig_ref[0] * (s - rhs_ref[...])
    out_ref[...] = jnp.where(y >= 0, y, 0.01 * y)


def _final(dual, rhs, spmm2, w1t, b1, sig):
    return pl.pallas_call(
        _final_body,
        grid=(N // BLK,),
        in_specs=[
            pl.BlockSpec((BLK, H), lambda i: (i, 0)),
            pl.BlockSpec((BLK, H), lambda i: (i, 0)),
            pl.BlockSpec((2, BLK, HH), lambda i: (0, i, 0)),
            pl.BlockSpec((H, H), lambda i: (0, 0)),
            pl.BlockSpec((1, H), lambda i: (0, 0)),
            pl.BlockSpec(memory_space=pltpu.SMEM),
        ],
        out_specs=pl.BlockSpec((BLK, H), lambda i: (i, 0)),
        out_shape=jax.ShapeDtypeStruct((N, H), jnp.float32),
    )(dual, rhs, spmm2, w1t, b1, sig)


def _sc_spmm_body(
    xs_hbm, cols_hbm, rows_hbm, vals_hbm, out_hbm,
    xs_sh, acc_sh, gb0, gb1, t_v, cols_v, rows_v, vals_v, gsem, ssem, isem,
):
    c = lax.axis_index("c")
    s = lax.axis_index("s")
    row0 = s * RT_OUT

    pltpu.sync_copy(
        xs_hbm.at[c, pl.ds(row0, RT_OUT)], xs_sh.at[pl.ds(row0, RT_OUT)]
    )

    zeros16 = jnp.zeros((16,), jnp.float32)

    def _zero_row(r, carry):
        t_v[r, pl.ds(0, 16)] = zeros16
        t_v[r, pl.ds(16, 16)] = zeros16
        return carry

    lax.fori_loop(0, SE, _zero_row, 0)
    for k in range(RT_OUT // SE):
        pltpu.sync_copy(t_v, acc_sh.at[pl.ds(row0 + k * SE, SE)])
    plsc.subcore_barrier()

    i_consts = [jnp.full((16,), i, jnp.int32) for i in range(16)]

    def _idx_copies(m, par):
        return (
            pltpu.make_async_copy(cols_hbm.at[s, m], cols_v.at[par], isem),
            pltpu.make_async_copy(rows_hbm.at[s, m], rows_v.at[par], isem),
            pltpu.make_async_copy(vals_hbm.at[s, m], vals_v.at[par], isem),
        )

    def _step(src_gb, dst_gb, m, par, drain_pred, fire_pred):
        @pl.when(drain_pred)
        def _drain_scatter():
            pltpu.make_async_copy(
                t_v, acc_sh.at[rows_v.at[1 - par]], ssem
            ).wait()

        @pl.when(fire_pred)
        def _fire_idx():
            for cp in _idx_copies(m + 1, 1 - par):
                cp.start()

        pltpu.make_async_copy(
            xs_sh.at[cols_v.at[par]], src_gb, gsem
        ).wait()

        vrow = vals_v.at[par]

        @plsc.parallel_loop(0, SE // 16, 1, unroll=2)
        def _mul16(k, vrow=vrow):
            ev = jnp.full((16,), k * 16, jnp.int32)
            for i in range(16):
                e = k * 16 + i
                vb = plsc.load_gather(vrow, [ev + i_consts[i]])
                for h in range(HH // 16):
                    t_v[e, pl.ds(h * 16, 16)] = (
                        src_gb[e, pl.ds(h * 16, 16)] * vb
                    )

        pltpu.async_copy(t_v, acc_sh.at[rows_v.at[par]], ssem, add=True)

        @pl.when(fire_pred)
        def _fire_gather():
            for cp in _idx_copies(m + 1, 1 - par):
                cp.wait()
            pltpu.async_copy(xs_sh.at[cols_v.at[1 - par]], dst_gb, gsem)

    for cp in _idx_copies(0, 0):
        cp.start()
    for cp in _idx_copies(0, 0):
        cp.wait()
    pltpu.async_copy(xs_sh.at[cols_v.at[0]], gb0, gsem)

    true_p = jnp.bool_(True)

    def _two_steps(m2, carry):
        m = m2 * 2
        _step(gb0, gb1, m, 0, m2 > 0, true_p)
        _step(gb1, gb0, m + 1, 1, true_p, m + 2 < NSTEP)
        return carry

    lax.fori_loop(0, NSTEP // 2, _two_steps, 0)
    pltpu.make_async_copy(t_v, acc_sh.at[rows_v.at[1]], ssem).wait()

    plsc.subcore_barrier()
    pltpu.sync_copy(
        acc_sh.at[pl.ds(row0, RT_OUT)], out_hbm.at[c, pl.ds(row0, RT_OUT)]
    )


@functools.cache
def _sc_spmm():
    return pl.kernel(
        _sc_spmm_body,
        out_type=jax.ShapeDtypeStruct((2, N, HH), jnp.float32),
        mesh=plsc.VectorSubcoreMesh(core_axis_name="c", subcore_axis_name="s"),
        compiler_params=pltpu.CompilerParams(
            needs_layout_passes=False, use_tc_tiling_on_sc=False
        ),
        scratch_types=[
            pltpu.VMEM_SHARED((N, HH), jnp.float32),
            pltpu.VMEM_SHARED((N, HH), jnp.float32),
            pltpu.VMEM((SE, HH), jnp.float32),
            pltpu.VMEM((SE, HH), jnp.float32),
            pltpu.VMEM((SE, HH), jnp.float32),
            pltpu.VMEM((2, SE), jnp.int32),
            pltpu.VMEM((2, SE), jnp.int32),
            pltpu.VMEM((2, SE), jnp.float32),
            pltpu.SemaphoreType.DMA,
            pltpu.SemaphoreType.DMA,
            pltpu.SemaphoreType.DMA,
        ],
    )


def kernel(primal, last_primal, dual, cons_indices, cons_values,
           right_hand_side, W1, b1, W2, b2, W3, b3, sigma):
    rows = cons_indices[0]
    cols = cons_indices[1]
    pad = NNZ_PAD - NNZ
    cols3 = jnp.pad(cols, (0, pad)).reshape(NS, NSTEP, SE)
    rows3 = jnp.pad(rows, (0, pad)).reshape(NS, NSTEP, SE)
    vals3 = jnp.pad(cons_values, (0, pad)).reshape(NS, NSTEP, SE)

    xs = _theta_diff(primal, last_primal, W2.T, W3.T, (b2 - b3).reshape(1, H))
    spmm2 = _sc_spmm()(xs, cols3, rows3, vals3)
    return _final(
        dual, right_hand_side, spmm2, W1.T, b1.reshape(1, H), sigma.reshape(1)
    )


# speedup vs baseline: 4.3848x; 1.1468x over previous
import functools

import jax
import jax.numpy as jnp
from jax import lax
from jax.experimental import pallas as pl
from jax.experimental.pallas import tpu as pltpu
from jax.experimental.pallas import tpu_sc as plsc

N = 16384
H = 64
HH = 32
NNZ = 2684354
NS = 16
SE = 512
RPT = 1312
NSTEP = RPT * 128 // SE
NNZ_PAD = NS * RPT * 128
RT_OUT = N // NS
BLK = 1024


def _interleave_half(yb, lo):
    a = yb[:, lo:lo + HH // 2]
    b = yb[:, lo + HH // 2:lo + HH]
    return jnp.stack([a, b], axis=-1).reshape(yb.shape[0], HH)


def _theta_diff_body(p_ref, lp_ref, w2t_ref, w3t_ref, bd_ref, out_ref):
    y = (
        jnp.dot(p_ref[...], w2t_ref[...], preferred_element_type=jnp.float32)
        - jnp.dot(lp_ref[...], w3t_ref[...], preferred_element_type=jnp.float32)
        + bd_ref[...]
    )
    yb = y.astype(jnp.bfloat16)
    out_ref[0] = _interleave_half(yb, 0)
    out_ref[1] = _interleave_half(yb, HH)


def _theta_diff(primal, last_primal, w2t, w3t, bd):
    return pl.pallas_call(
        _theta_diff_body,
        grid=(N // BLK,),
        in_specs=[
            pl.BlockSpec((BLK, H), lambda i: (i, 0)),
            pl.BlockSpec((BLK, H), lambda i: (i, 0)),
            pl.BlockSpec((H, H), lambda i: (0, 0)),
            pl.BlockSpec((H, H), lambda i: (0, 0)),
            pl.BlockSpec((1, H), lambda i: (0, 0)),
        ],
        out_specs=pl.BlockSpec((2, BLK, HH), lambda i: (0, i, 0)),
        out_shape=jax.ShapeDtypeStruct((2, N, HH), jnp.bfloat16),
    )(primal, last_primal, w2t, w3t, bd)


def _final_body(d_ref, rhs_ref, sp_ref, w1t_ref, b1_ref, sig_ref, out_ref):
    y = (
        jnp.dot(d_ref[...], w1t_ref[...], preferred_element_type=jnp.float32)
        + b1_ref[...]
    )
    s = jnp.concatenate([sp_ref[0], sp_ref[1]], axis=1)
    y = y + sig_ref[0] * (s - rhs_ref[...])
    out_ref[...] = jnp.where(y >= 0, y, 0.01 * y)


def _final(dual, rhs, spmm2, w1t, b1, sig):
    return pl.pallas_call(
        _final_body,
        grid=(N // BLK,),
        in_specs=[
            pl.BlockSpec((BLK, H), lambda i: (i, 0)),
            pl.BlockSpec((BLK, H), lambda i: (i, 0)),
            pl.BlockSpec((2, BLK, HH), lambda i: (0, i, 0)),
            pl.BlockSpec((H, H), lambda i: (0, 0)),
            pl.BlockSpec((1, H), lambda i: (0, 0)),
            pl.BlockSpec(memory_space=pltpu.SMEM),
        ],
        out_specs=pl.BlockSpec((BLK, H), lambda i: (i, 0)),
        out_shape=jax.ShapeDtypeStruct((N, H), jnp.float32),
    )(dual, rhs, spmm2, w1t, b1, sig)


def _sc_spmm_body(
    xs_hbm, cols_hbm, rows_hbm, vals_hbm, out_hbm,
    xs_sh, acc_sh, gb0, gb1, t_v, cols_v, rows_v, vals_v, gsem, ssem, isem,
):
    c = lax.axis_index("c")
    s = lax.axis_index("s")
    row0 = s * RT_OUT

    pltpu.sync_copy(
        xs_hbm.at[c, pl.ds(row0, RT_OUT)], xs_sh.at[pl.ds(row0, RT_OUT)]
    )

    zeros16 = jnp.zeros((16,), jnp.float32)

    def _zero_row(r, carry):
        t_v[r, pl.ds(0, 16)] = zeros16
        t_v[r, pl.ds(16, 16)] = zeros16
        return carry

    lax.fori_loop(0, SE, _zero_row, 0)
    for k in range(RT_OUT // SE):
        pltpu.sync_copy(t_v, acc_sh.at[pl.ds(row0 + k * SE, SE)])
    plsc.subcore_barrier()

    i_consts = [jnp.full((16,), i, jnp.int32) for i in range(16)]

    def _idx_copies(m, par):
        return (
            pltpu.make_async_copy(cols_hbm.at[s, m], cols_v.at[par], isem),
            pltpu.make_async_copy(rows_hbm.at[s, m], rows_v.at[par], isem),
            pltpu.make_async_copy(vals_hbm.at[s, m], vals_v.at[par], isem),
        )

    def _step(src_gb, dst_gb, m, par, drain_pred, fire_pred):
        @pl.when(drain_pred)
        def _drain_scatter():
            pltpu.make_async_copy(
                t_v, acc_sh.at[rows_v.at[1 - par]], ssem
            ).wait()

        @pl.when(fire_pred)
        def _fire_idx():
            for cp in _idx_copies(m + 1, 1 - par):
                cp.start()

        pltpu.make_async_copy(
            xs_sh.at[cols_v.at[par]], src_gb, gsem
        ).wait()

        vrow = vals_v.at[par]

        @plsc.parallel_loop(0, SE // 16, 1, unroll=2)
        def _mul16(k, vrow=vrow):
            ev = jnp.full((16,), k * 16, jnp.int32)
            for i in range(16):
                e = k * 16 + i
                vb = plsc.load_gather(vrow, [ev + i_consts[i]])
                g = src_gb[e, pl.ds(0, HH)]
                a, b = plsc.unpack(g, format=plsc.PackFormat.INTERLEAVED)
                t_v[e, pl.ds(0, 16)] = a * vb
                t_v[e, pl.ds(16, 16)] = b * vb

        pltpu.async_copy(t_v, acc_sh.at[rows_v.at[par]], ssem, add=True)

        @pl.when(fire_pred)
        def _fire_gather():
            for cp in _idx_copies(m + 1, 1 - par):
                cp.wait()
            pltpu.async_copy(xs_sh.at[cols_v.at[1 - par]], dst_gb, gsem)

    for cp in _idx_copies(0, 0):
        cp.start()
    for cp in _idx_copies(0, 0):
        cp.wait()
    pltpu.async_copy(xs_sh.at[cols_v.at[0]], gb0, gsem)

    true_p = jnp.bool_(True)

    def _two_steps(m2, carry):
        m = m2 * 2
        _step(gb0, gb1, m, 0, m2 > 0, true_p)
        _step(gb1, gb0, m + 1, 1, true_p, m + 2 < NSTEP)
        return carry

    lax.fori_loop(0, NSTEP // 2, _two_steps, 0)
    pltpu.make_async_copy(t_v, acc_sh.at[rows_v.at[1]], ssem).wait()

    plsc.subcore_barrier()
    pltpu.sync_copy(
        acc_sh.at[pl.ds(row0, RT_OUT)], out_hbm.at[c, pl.ds(row0, RT_OUT)]
    )


@functools.cache
def _sc_spmm():
    return pl.kernel(
        _sc_spmm_body,
        out_type=jax.ShapeDtypeStruct((2, N, HH), jnp.float32),
        mesh=plsc.VectorSubcoreMesh(core_axis_name="c", subcore_axis_name="s"),
        compiler_params=pltpu.CompilerParams(
            needs_layout_passes=False, use_tc_tiling_on_sc=False
        ),
        scratch_types=[
            pltpu.VMEM_SHARED((N, HH), jnp.bfloat16),
            pltpu.VMEM_SHARED((N, HH), jnp.float32),
            pltpu.VMEM((SE, HH), jnp.bfloat16),
            pltpu.VMEM((SE, HH), jnp.bfloat16),
            pltpu.VMEM((SE, HH), jnp.float32),
            pltpu.VMEM((2, SE), jnp.int32),
            pltpu.VMEM((2, SE), jnp.int32),
            pltpu.VMEM((2, SE), jnp.float32),
            pltpu.SemaphoreType.DMA,
            pltpu.SemaphoreType.DMA,
            pltpu.SemaphoreType.DMA,
        ],
    )


def kernel(primal, last_primal, dual, cons_indices, cons_values,
           right_hand_side, W1, b1, W2, b2, W3, b3, sigma):
    rows = cons_indices[0]
    cols = cons_indices[1]
    pad = NNZ_PAD - NNZ
    cols3 = jnp.pad(cols, (0, pad)).reshape(NS, NSTEP, SE)
    rows3 = jnp.pad(rows, (0, pad)).reshape(NS, NSTEP, SE)
    vals3 = jnp.pad(cons_values, (0, pad)).reshape(NS, NSTEP, SE)

    xs = _theta_diff(primal, last_primal, W2.T, W3.T, (b2 - b3).reshape(1, H))
    spmm2 = _sc_spmm()(xs, cols3, rows3, vals3)
    return _final(
        dual, right_hand_side, spmm2, W1.T, b1.reshape(1, H), sigma.reshape(1)
    )
